# baseline (speedup 1.0000x reference)
import functools

import jax
import jax.numpy as jnp
from jax import lax
from jax.experimental import pallas as pl
from jax.experimental.pallas import tpu as pltpu
from jax.experimental.pallas import tpu_sc as plsc

_D = 768
_LANES = 16
_NC = 2
_NS = 16
_NW = _NC * _NS


@functools.lru_cache(maxsize=None)
def _emb_kernel(n_rows: int, rows_pw: int, chunk: int):
    mesh = plsc.VectorSubcoreMesh(core_axis_name="c", subcore_axis_name="s")
    n_chunks = rows_pw // chunk
    n_slices = _D // _LANES

    @functools.partial(
        pl.kernel, mesh=mesh,
        out_type=jax.ShapeDtypeStruct((n_rows, _D), jnp.float32),
        scratch_types=[
            pltpu.VMEM((rows_pw,), jnp.int32),
            pltpu.VMEM((rows_pw,), jnp.int32),
            pltpu.VMEM((rows_pw,), jnp.int32),
            pltpu.VMEM((chunk, _D), jnp.float32),
            pltpu.VMEM((chunk, _D), jnp.float32),
            pltpu.VMEM((chunk, _D), jnp.float32),
            pltpu.SemaphoreType.DMA,
        ],
    )
    def body(iw_hbm, ip_hbm, it_hbm, wt_hbm, pt_hbm, tt_hbm, out_hbm,
             iw_v, ip_v, it_v, w_v, p_v, t_v, sem):
        wid = lax.axis_index("s") * _NC + lax.axis_index("c")
        base = wid * rows_pw
        pltpu.sync_copy(iw_hbm.at[pl.ds(base, rows_pw)], iw_v)
        pltpu.sync_copy(ip_hbm.at[pl.ds(base, rows_pw)], ip_v)
        pltpu.sync_copy(it_hbm.at[pl.ds(base, rows_pw)], it_v)

        def do_chunk(k, carry):
            off = k * chunk
            cw = pltpu.async_copy(wt_hbm.at[iw_v.at[pl.ds(off, chunk)]], w_v, sem)
            cp = pltpu.async_copy(pt_hbm.at[ip_v.at[pl.ds(off, chunk)]], p_v, sem)
            ct = pltpu.async_copy(tt_hbm.at[it_v.at[pl.ds(off, chunk)]], t_v, sem)
            cw.wait()
            cp.wait()
            ct.wait()

            def do_row(r, carry2):
                for j in range(n_slices):
                    s = pl.ds(j * _LANES, _LANES)
                    w_v[r, s] = w_v[r, s] + p_v[r, s] + t_v[r, s]
                return carry2

            lax.fori_loop(0, chunk, do_row, 0)
            pltpu.sync_copy(w_v, out_hbm.at[pl.ds(base + off, chunk)])
            return carry

        lax.fori_loop(0, n_chunks, do_chunk, 0)

    return body


def kernel(input_ids, position_ids, token_type_ids, word_embeddings,
           position_embeddings, token_type_embeddings):
    b, s = input_ids.shape
    n_rows = b * s
    iw = input_ids.reshape(n_rows).astype(jnp.int32)
    ip = position_ids.reshape(n_rows).astype(jnp.int32)
    it = token_type_ids.reshape(n_rows).astype(jnp.int32)
    rows_pw = n_rows // _NW
    k = _emb_kernel(n_rows, rows_pw, chunk=32)
    out = k(iw, ip, it, word_embeddings, position_embeddings,
            token_type_embeddings)
    return out.reshape(b, s, _D)
